# baseline (speedup 1.0000x reference)
import functools

import jax
import jax.numpy as jnp
from jax import lax
from jax.experimental import pallas as pl
from jax.experimental.pallas import tpu as pltpu
from jax.experimental.pallas import tpu_sc as plsc

_NC = 2
_NS = 16
_NW = _NC * _NS

_RB = 2000
_CH = 80
_HALVES = 5
_DW = 16


def _segsum_sc(n, e, d, with_deg=False):
  assert e % (_NW * _CH * _HALVES) == 0
  chunks_per_worker = e // (_NW * _CH)
  cpw_h = chunks_per_worker // _HALVES
  assert n % _NS == 0
  rows_per_tile = n // _NS
  zr = rows_per_tile // 5
  assert rows_per_tile == 5 * zr

  mesh = plsc.VectorSubcoreMesh(core_axis_name="c", subcore_axis_name="s")

  out_type = [jax.ShapeDtypeStruct((_NC, n, d), jnp.float32)]
  scratch = [
      pltpu.VMEM((cpw_h * _CH,), jnp.int32),
      pltpu.VMEM((cpw_h * _CH,), jnp.int32),
      pltpu.VMEM((_CH, d), jnp.float32),
      pltpu.VMEM((_CH, d), jnp.float32),
      pltpu.VMEM_SHARED((n, d), jnp.float32),
      pltpu.SemaphoreType.DMA,
      pltpu.SemaphoreType.DMA,
  ]
  if with_deg:
    out_type.append(jax.ShapeDtypeStruct((_NC, n, _DW), jnp.float32))
    scratch += [
        pltpu.VMEM((_CH, _DW), jnp.float32),
        pltpu.VMEM_SHARED((n, _DW), jnp.float32),
    ]

  @functools.partial(
      pl.kernel,
      mesh=mesh,
      compiler_params=pltpu.CompilerParams(use_tc_tiling_on_sc=False),
      out_type=out_type,
      scratch_types=scratch,
  )
  def seg(*refs):
    if with_deg:
      (table_hbm, edge_hbm, zeros_hbm, aux_hbm,
       out_hbm, deg_hbm, src_v, dst_v, rows_a, rows_b, acc_sh,
       sem_a, sem_b, ones_v, dacc_sh) = refs
    else:
      (table_hbm, edge_hbm, zeros_hbm,
       out_hbm, src_v, dst_v, rows_a, rows_b, acc_sh, sem_a, sem_b) = refs
    cid = lax.axis_index("c")
    sid = lax.axis_index("s")
    wid = cid * _NS + sid

    r0 = sid * rows_per_tile

    def zbody(z, carry):
      pltpu.sync_copy(zeros_hbm, acc_sh.at[pl.ds(r0 + z * zr, zr)])
      if with_deg:
        pltpu.sync_copy(aux_hbm.at[pl.ds(0, zr)],
                        dacc_sh.at[pl.ds(r0 + z * zr, zr)])
      return carry

    lax.fori_loop(0, 5, zbody, 0)
    if with_deg:
      pltpu.sync_copy(aux_hbm.at[pl.ds(128, _CH)], ones_v)
    plsc.subcore_barrier()

    tmax = cpw_h - 1

    def gather(t, buf, sem):
      pltpu.async_copy(table_hbm.at[src_v.at[pl.ds(t * _CH, _CH)]], buf, sem)

    def drain(buf, sem):
      pltpu.make_async_copy(table_hbm.at[src_v.at[pl.ds(0, _CH)]],
                            buf, sem).wait()

    def scat(t, buf):
      pltpu.sync_copy(buf, acc_sh.at[dst_v.at[pl.ds(t * _CH, _CH)]], add=True)
      if with_deg:
        pltpu.sync_copy(ones_v,
                        dacc_sh.at[dst_v.at[pl.ds(t * _CH, _CH)]], add=True)

    epw = chunks_per_worker * _CH
    for half in range(_HALVES):
      off = wid * epw + half * (cpw_h * _CH)
      pltpu.sync_copy(edge_hbm.at[0, pl.ds(off, cpw_h * _CH)], src_v)
      pltpu.sync_copy(edge_hbm.at[1, pl.ds(off, cpw_h * _CH)], dst_v)
      gather(0, rows_a, sem_a)

      def body(i, carry):
        t0 = 2 * i
        gather(t0 + 1, rows_b, sem_b)
        drain(rows_a, sem_a)
        scat(t0, rows_a)
        gather(jnp.minimum(t0 + 2, tmax), rows_a, sem_a)
        drain(rows_b, sem_b)
        scat(t0 + 1, rows_b)
        return carry

      lax.fori_loop(0, cpw_h // 2, body, 0)
      if cpw_h % 2:
        drain(rows_a, sem_a)
        scat(tmax, rows_a)
      else:
        drain(rows_a, sem_a)

    plsc.subcore_barrier()

    def wbody(z, carry):
      rr = r0 + z * zr
      pltpu.sync_copy(acc_sh.at[pl.ds(rr, zr)], out_hbm.at[cid, pl.ds(rr, zr)])
      if with_deg:
        pltpu.sync_copy(dacc_sh.at[pl.ds(rr, zr)],
                        deg_hbm.at[cid, pl.ds(rr, zr)])
      return carry

    lax.fori_loop(0, 5, wbody, 0)

  return seg


def _mid_tc(n, f, h, c):
  def body(x_ref, parts_ref, degp_ref, b1_ref, ws1_ref, wn1_ref,
           wn2_ref, ws2_ref, p2_ref, hs2_ref):
    deg = degp_ref[0, :, :1] + degp_ref[1, :, :1]
    dinv = 1.0 / jnp.maximum(deg, 1.0)
    hn = (parts_ref[0] + parts_ref[1]) * dinv
    h1 = jnp.dot(x_ref[...], ws1_ref[...], preferred_element_type=jnp.float32)
    h1 += jnp.dot(hn, wn1_ref[...], preferred_element_type=jnp.float32)
    h1 = jnp.maximum(h1 + b1_ref[0], 0.0)
    p2_ref[...] = jnp.dot(h1, wn2_ref[...], preferred_element_type=jnp.float32)
    hs2_ref[...] = jnp.dot(h1, ws2_ref[...], preferred_element_type=jnp.float32)

  return pl.pallas_call(
      body,
      grid=(n // _RB,),
      in_specs=[
          pl.BlockSpec((_RB, f), lambda i: (i, 0)),
          pl.BlockSpec((_NC, _RB, f), lambda i: (0, i, 0)),
          pl.BlockSpec((_NC, _RB, _DW), lambda i: (0, i, 0)),
          pl.BlockSpec((1, h), lambda i: (0, 0)),
          pl.BlockSpec((f, h), lambda i: (0, 0)),
          pl.BlockSpec((f, h), lambda i: (0, 0)),
          pl.BlockSpec((h, c), lambda i: (0, 0)),
          pl.BlockSpec((h, c), lambda i: (0, 0)),
      ],
      out_specs=[
          pl.BlockSpec((_RB, c), lambda i: (i, 0)),
          pl.BlockSpec((_RB, c), lambda i: (i, 0)),
      ],
      out_shape=[
          jax.ShapeDtypeStruct((n, c), jnp.float32),
          jax.ShapeDtypeStruct((n, c), jnp.float32),
      ],
  )


def _final_tc(n, c):
  def body(hs_ref, q_ref, degp_ref, b2_ref, out_ref):
    deg = degp_ref[0, :, :1] + degp_ref[1, :, :1]
    dinv = 1.0 / jnp.maximum(deg, 1.0)
    out_ref[...] = (hs_ref[...]
                    + (q_ref[0] + q_ref[1]) * dinv
                    + b2_ref[0])

  return pl.pallas_call(
      body,
      grid=(n // _RB,),
      in_specs=[
          pl.BlockSpec((_RB, c), lambda i: (i, 0)),
          pl.BlockSpec((_NC, _RB, c), lambda i: (0, i, 0)),
          pl.BlockSpec((_NC, _RB, _DW), lambda i: (0, i, 0)),
          pl.BlockSpec((1, c), lambda i: (0, 0)),
      ],
      out_specs=pl.BlockSpec((_RB, c), lambda i: (i, 0)),
      out_shape=jax.ShapeDtypeStruct((n, c), jnp.float32),
  )


@jax.jit
def kernel(in_feat, edge_index, W_self1, W_neigh1, b1, W_self2, W_neigh2, b2):
  n, f = in_feat.shape
  h = W_self1.shape[1]
  c = W_self2.shape[1]
  e = edge_index.shape[1]

  zr = n // _NS // 5
  zeros1 = jnp.zeros((zr, f), jnp.float32)
  zeros2 = jnp.zeros((zr, c), jnp.float32)
  aux = jnp.concatenate([jnp.zeros((128, _DW), jnp.float32),
                         jnp.ones((_CH, _DW), jnp.float32)])

  parts1, degp = _segsum_sc(n, e, f, with_deg=True)(
      in_feat, edge_index, zeros1, aux)
  p2, hs2 = _mid_tc(n, f, h, c)(
      in_feat, parts1, degp, b1.reshape(1, h), W_self1, W_neigh1,
      W_neigh2, W_self2)
  parts2, = _segsum_sc(n, e, c)(p2, edge_index, zeros2)
  return _final_tc(n, c)(hs2, parts2, degp, b2.reshape(1, c))


# speedup vs baseline: 1.0425x; 1.0425x over previous
import functools

import jax
import jax.numpy as jnp
from jax import lax
from jax.experimental import pallas as pl
from jax.experimental.pallas import tpu as pltpu
from jax.experimental.pallas import tpu_sc as plsc

_NC = 2
_NS = 16
_NW = _NC * _NS

_RB = 2000
_CH = 128
_HALVES = 6
_DW = 16


def _segsum_sc(n, e, d, with_deg=False):
  assert e % _CH == 0
  chunks_per_worker = e // (_NW * _CH)
  nextra = e // _CH - _NW * chunks_per_worker
  assert nextra < _NW
  assert chunks_per_worker % _HALVES == 0
  cpw_h = chunks_per_worker // _HALVES
  assert n % _NS == 0
  rows_per_tile = n // _NS
  zr = rows_per_tile // 5
  assert rows_per_tile == 5 * zr

  mesh = plsc.VectorSubcoreMesh(core_axis_name="c", subcore_axis_name="s")

  out_type = [jax.ShapeDtypeStruct((_NC, n, d), jnp.float32)]
  scratch = [
      pltpu.VMEM((cpw_h * _CH,), jnp.int32),
      pltpu.VMEM((cpw_h * _CH,), jnp.int32),
      pltpu.VMEM((_CH, d), jnp.float32),
      pltpu.VMEM((_CH, d), jnp.float32),
      pltpu.VMEM_SHARED((n, d), jnp.float32),
      pltpu.SemaphoreType.DMA,
      pltpu.SemaphoreType.DMA,
  ]
  if with_deg:
    out_type.append(jax.ShapeDtypeStruct((_NC, n, _DW), jnp.float32))
    scratch += [
        pltpu.VMEM((_CH, _DW), jnp.float32),
        pltpu.VMEM_SHARED((n, _DW), jnp.float32),
    ]

  @functools.partial(
      pl.kernel,
      mesh=mesh,
      compiler_params=pltpu.CompilerParams(use_tc_tiling_on_sc=False),
      out_type=out_type,
      scratch_types=scratch,
  )
  def seg(*refs):
    if with_deg:
      (table_hbm, edge_hbm, zeros_hbm, aux_hbm,
       out_hbm, deg_hbm, src_v, dst_v, rows_a, rows_b, acc_sh,
       sem_a, sem_b, ones_v, dacc_sh) = refs
    else:
      (table_hbm, edge_hbm, zeros_hbm,
       out_hbm, src_v, dst_v, rows_a, rows_b, acc_sh, sem_a, sem_b) = refs
    cid = lax.axis_index("c")
    sid = lax.axis_index("s")
    wid = cid * _NS + sid

    r0 = sid * rows_per_tile

    def zbody(z, carry):
      pltpu.sync_copy(zeros_hbm, acc_sh.at[pl.ds(r0 + z * zr, zr)])
      if with_deg:
        pltpu.sync_copy(aux_hbm.at[pl.ds(0, zr)],
                        dacc_sh.at[pl.ds(r0 + z * zr, zr)])
      return carry

    lax.fori_loop(0, 5, zbody, 0)
    if with_deg:
      pltpu.sync_copy(aux_hbm.at[pl.ds(128, _CH)], ones_v)
    plsc.subcore_barrier()

    tmax = cpw_h - 1

    def gather(t, buf, sem):
      pltpu.async_copy(table_hbm.at[src_v.at[pl.ds(t * _CH, _CH)]], buf, sem)

    def drain(buf, sem):
      pltpu.make_async_copy(table_hbm.at[src_v.at[pl.ds(0, _CH)]],
                            buf, sem).wait()

    def scat(t, buf):
      pltpu.sync_copy(buf, acc_sh.at[dst_v.at[pl.ds(t * _CH, _CH)]], add=True)
      if with_deg:
        pltpu.sync_copy(ones_v,
                        dacc_sh.at[dst_v.at[pl.ds(t * _CH, _CH)]], add=True)

    epw = chunks_per_worker * _CH
    for half in range(_HALVES):
      off = wid * epw + half * (cpw_h * _CH)
      pltpu.sync_copy(edge_hbm.at[0, pl.ds(off, cpw_h * _CH)], src_v)
      pltpu.sync_copy(edge_hbm.at[1, pl.ds(off, cpw_h * _CH)], dst_v)
      gather(0, rows_a, sem_a)

      def body(i, carry):
        t0 = 2 * i
        gather(t0 + 1, rows_b, sem_b)
        drain(rows_a, sem_a)
        scat(t0, rows_a)
        gather(jnp.minimum(t0 + 2, tmax), rows_a, sem_a)
        drain(rows_b, sem_b)
        scat(t0 + 1, rows_b)
        return carry

      lax.fori_loop(0, cpw_h // 2, body, 0)
      if cpw_h % 2:
        drain(rows_a, sem_a)
        scat(tmax, rows_a)
      else:
        drain(rows_a, sem_a)

    if nextra:
      @pl.when(wid < nextra)
      def _():
        off2 = chunks_per_worker * _NW * _CH + wid * _CH
        pltpu.sync_copy(edge_hbm.at[0, pl.ds(off2, _CH)],
                        src_v.at[pl.ds(0, _CH)])
        pltpu.sync_copy(edge_hbm.at[1, pl.ds(off2, _CH)],
                        dst_v.at[pl.ds(0, _CH)])
        gather(0, rows_a, sem_a)
        drain(rows_a, sem_a)
        scat(0, rows_a)

    plsc.subcore_barrier()

    def wbody(z, carry):
      rr = r0 + z * zr
      pltpu.sync_copy(acc_sh.at[pl.ds(rr, zr)], out_hbm.at[cid, pl.ds(rr, zr)])
      if with_deg:
        pltpu.sync_copy(dacc_sh.at[pl.ds(rr, zr)],
                        deg_hbm.at[cid, pl.ds(rr, zr)])
      return carry

    lax.fori_loop(0, 5, wbody, 0)

  return seg


def _mid_tc(n, f, h, c):
  def body(x_ref, parts_ref, degp_ref, b1_ref, ws1_ref, wn1_ref,
           wn2_ref, ws2_ref, p2_ref, hs2_ref):
    deg = degp_ref[0, :, :1] + degp_ref[1, :, :1]
    dinv = 1.0 / jnp.maximum(deg, 1.0)
    hn = (parts_ref[0] + parts_ref[1]) * dinv
    h1 = jnp.dot(x_ref[...], ws1_ref[...], preferred_element_type=jnp.float32)
    h1 += jnp.dot(hn, wn1_ref[...], preferred_element_type=jnp.float32)
    h1 = jnp.maximum(h1 + b1_ref[0], 0.0)
    p2_ref[...] = jnp.dot(h1, wn2_ref[...], preferred_element_type=jnp.float32)
    hs2_ref[...] = jnp.dot(h1, ws2_ref[...], preferred_element_type=jnp.float32)

  return pl.pallas_call(
      body,
      grid=(n // _RB,),
      in_specs=[
          pl.BlockSpec((_RB, f), lambda i: (i, 0)),
          pl.BlockSpec((_NC, _RB, f), lambda i: (0, i, 0)),
          pl.BlockSpec((_NC, _RB, _DW), lambda i: (0, i, 0)),
          pl.BlockSpec((1, h), lambda i: (0, 0)),
          pl.BlockSpec((f, h), lambda i: (0, 0)),
          pl.BlockSpec((f, h), lambda i: (0, 0)),
          pl.BlockSpec((h, c), lambda i: (0, 0)),
          pl.BlockSpec((h, c), lambda i: (0, 0)),
      ],
      out_specs=[
          pl.BlockSpec((_RB, c), lambda i: (i, 0)),
          pl.BlockSpec((_RB, c), lambda i: (i, 0)),
      ],
      out_shape=[
          jax.ShapeDtypeStruct((n, c), jnp.float32),
          jax.ShapeDtypeStruct((n, c), jnp.float32),
      ],
  )


def _final_tc(n, c):
  def body(hs_ref, q_ref, degp_ref, b2_ref, out_ref):
    deg = degp_ref[0, :, :1] + degp_ref[1, :, :1]
    dinv = 1.0 / jnp.maximum(deg, 1.0)
    out_ref[...] = (hs_ref[...]
                    + (q_ref[0] + q_ref[1]) * dinv
                    + b2_ref[0])

  return pl.pallas_call(
      body,
      grid=(n // _RB,),
      in_specs=[
          pl.BlockSpec((_RB, c), lambda i: (i, 0)),
          pl.BlockSpec((_NC, _RB, c), lambda i: (0, i, 0)),
          pl.BlockSpec((_NC, _RB, _DW), lambda i: (0, i, 0)),
          pl.BlockSpec((1, c), lambda i: (0, 0)),
      ],
      out_specs=pl.BlockSpec((_RB, c), lambda i: (i, 0)),
      out_shape=jax.ShapeDtypeStruct((n, c), jnp.float32),
  )


@jax.jit
def kernel(in_feat, edge_index, W_self1, W_neigh1, b1, W_self2, W_neigh2, b2):
  n, f = in_feat.shape
  h = W_self1.shape[1]
  c = W_self2.shape[1]
  e = edge_index.shape[1]

  zr = n // _NS // 5
  zeros1 = jnp.zeros((zr, f), jnp.float32)
  zeros2 = jnp.zeros((zr, c), jnp.float32)
  aux = jnp.concatenate([jnp.zeros((128, _DW), jnp.float32),
                         jnp.ones((_CH, _DW), jnp.float32)])

  parts1, degp = _segsum_sc(n, e, f, with_deg=True)(
      in_feat, edge_index, zeros1, aux)
  p2, hs2 = _mid_tc(n, f, h, c)(
      in_feat, parts1, degp, b1.reshape(1, h), W_self1, W_neigh1,
      W_neigh2, W_self2)
  parts2, = _segsum_sc(n, e, c)(p2, edge_index, zeros2)
  return _final_tc(n, c)(hs2, parts2, degp, b2.reshape(1, c))


# speedup vs baseline: 1.0570x; 1.0139x over previous
import functools

import jax
import jax.numpy as jnp
from jax import lax
from jax.experimental import pallas as pl
from jax.experimental.pallas import tpu as pltpu
from jax.experimental.pallas import tpu_sc as plsc

_NC = 2
_NS = 16
_NW = _NC * _NS

_RB = 2000
_CH = 128
_HALVES = 6
_DW = 16


def _segsum_sc(n, e, d, with_deg=False):
  assert e % _CH == 0
  chunks_per_worker = e // (_NW * _CH)
  nextra = e // _CH - _NW * chunks_per_worker
  assert nextra < _NW
  assert chunks_per_worker % _HALVES == 0
  cpw_h = chunks_per_worker // _HALVES
  assert n % _NS == 0
  rows_per_tile = n // _NS
  zr = rows_per_tile // 5
  assert rows_per_tile == 5 * zr

  mesh = plsc.VectorSubcoreMesh(core_axis_name="c", subcore_axis_name="s")

  out_type = [jax.ShapeDtypeStruct((_NC, n, d), jnp.float32)]
  scratch = [
      pltpu.VMEM((cpw_h * _CH,), jnp.int32),
      pltpu.VMEM((cpw_h * _CH,), jnp.int32),
      pltpu.VMEM((_CH, d), jnp.float32),
      pltpu.VMEM((_CH, d), jnp.float32),
      pltpu.VMEM_SHARED((n, d), jnp.float32),
      pltpu.SemaphoreType.DMA,
      pltpu.SemaphoreType.DMA,
  ]
  if with_deg:
    out_type.append(jax.ShapeDtypeStruct((_NC, n, _DW), jnp.float32))
    scratch += [
        pltpu.VMEM((_CH, _DW), jnp.float32),
        pltpu.VMEM_SHARED((n, _DW), jnp.float32),
        pltpu.SemaphoreType.DMA,
    ]

  @functools.partial(
      pl.kernel,
      mesh=mesh,
      compiler_params=pltpu.CompilerParams(use_tc_tiling_on_sc=False),
      out_type=out_type,
      scratch_types=scratch,
  )
  def seg(*refs):
    if with_deg:
      (table_hbm, edge_hbm, zeros_hbm, aux_hbm,
       out_hbm, deg_hbm, src_v, dst_v, rows_a, rows_b, acc_sh,
       sem_a, sem_b, ones_v, dacc_sh, osem) = refs
    else:
      (table_hbm, edge_hbm, zeros_hbm,
       out_hbm, src_v, dst_v, rows_a, rows_b, acc_sh, sem_a, sem_b) = refs
    cid = lax.axis_index("c")
    sid = lax.axis_index("s")
    wid = cid * _NS + sid

    r0 = sid * rows_per_tile

    def zbody(z, carry):
      pltpu.sync_copy(zeros_hbm, acc_sh.at[pl.ds(r0 + z * zr, zr)])
      if with_deg:
        pltpu.sync_copy(aux_hbm.at[pl.ds(0, zr)],
                        dacc_sh.at[pl.ds(r0 + z * zr, zr)])
      return carry

    lax.fori_loop(0, 5, zbody, 0)
    if with_deg:
      pltpu.sync_copy(aux_hbm.at[pl.ds(128, _CH)], ones_v)
    plsc.subcore_barrier()

    tmax = cpw_h - 1

    def gather(t, buf, sem):
      pltpu.async_copy(table_hbm.at[src_v.at[pl.ds(t * _CH, _CH)]], buf, sem)

    def drain(buf, sem):
      pltpu.make_async_copy(table_hbm.at[src_v.at[pl.ds(0, _CH)]],
                            buf, sem).wait()

    def scat(t, buf):
      pltpu.sync_copy(buf, acc_sh.at[dst_v.at[pl.ds(t * _CH, _CH)]], add=True)
      if with_deg:
        pltpu.async_copy(ones_v,
                         dacc_sh.at[dst_v.at[pl.ds(t * _CH, _CH)]], osem,
                         add=True)

    epw = chunks_per_worker * _CH
    for half in range(_HALVES):
      off = wid * epw + half * (cpw_h * _CH)
      pltpu.sync_copy(edge_hbm.at[0, pl.ds(off, cpw_h * _CH)], src_v)
      pltpu.sync_copy(edge_hbm.at[1, pl.ds(off, cpw_h * _CH)], dst_v)
      gather(0, rows_a, sem_a)

      def body(i, carry):
        t0 = 2 * i
        gather(t0 + 1, rows_b, sem_b)
        drain(rows_a, sem_a)
        scat(t0, rows_a)
        gather(jnp.minimum(t0 + 2, tmax), rows_a, sem_a)
        drain(rows_b, sem_b)
        scat(t0 + 1, rows_b)
        return carry

      lax.fori_loop(0, cpw_h // 2, body, 0)
      if cpw_h % 2:
        drain(rows_a, sem_a)
        scat(tmax, rows_a)
      else:
        drain(rows_a, sem_a)

    if nextra:
      @pl.when(wid < nextra)
      def _():
        off2 = chunks_per_worker * _NW * _CH + wid * _CH
        pltpu.sync_copy(edge_hbm.at[0, pl.ds(off2, _CH)],
                        src_v.at[pl.ds(0, _CH)])
        pltpu.sync_copy(edge_hbm.at[1, pl.ds(off2, _CH)],
                        dst_v.at[pl.ds(0, _CH)])
        gather(0, rows_a, sem_a)
        drain(rows_a, sem_a)
        scat(0, rows_a)
        if with_deg:
          pltpu.make_async_copy(
              ones_v, dacc_sh.at[dst_v.at[pl.ds(0, _CH)]], osem).wait()

    if with_deg:
      def dbody(t, carry):
        pltpu.make_async_copy(
            ones_v, dacc_sh.at[dst_v.at[pl.ds(0, _CH)]], osem).wait()
        return carry

      lax.fori_loop(0, chunks_per_worker, dbody, 0)

    plsc.subcore_barrier()

    def wbody(z, carry):
      rr = r0 + z * zr
      pltpu.sync_copy(acc_sh.at[pl.ds(rr, zr)], out_hbm.at[cid, pl.ds(rr, zr)])
      if with_deg:
        pltpu.sync_copy(dacc_sh.at[pl.ds(rr, zr)],
                        deg_hbm.at[cid, pl.ds(rr, zr)])
      return carry

    lax.fori_loop(0, 5, wbody, 0)

  return seg


def _mid_tc(n, f, h, c):
  def body(x_ref, parts_ref, degp_ref, b1_ref, ws1_ref, wn1_ref,
           wn2_ref, ws2_ref, p2_ref, hs2_ref):
    deg = degp_ref[0, :, :1] + degp_ref[1, :, :1]
    dinv = 1.0 / jnp.maximum(deg, 1.0)
    hn = (parts_ref[0] + parts_ref[1]) * dinv
    h1 = jnp.dot(x_ref[...], ws1_ref[...], preferred_element_type=jnp.float32)
    h1 += jnp.dot(hn, wn1_ref[...], preferred_element_type=jnp.float32)
    h1 = jnp.maximum(h1 + b1_ref[0], 0.0)
    p2_ref[...] = jnp.dot(h1, wn2_ref[...], preferred_element_type=jnp.float32)
    hs2_ref[...] = jnp.dot(h1, ws2_ref[...], preferred_element_type=jnp.float32)

  return pl.pallas_call(
      body,
      grid=(n // _RB,),
      in_specs=[
          pl.BlockSpec((_RB, f), lambda i: (i, 0)),
          pl.BlockSpec((_NC, _RB, f), lambda i: (0, i, 0)),
          pl.BlockSpec((_NC, _RB, _DW), lambda i: (0, i, 0)),
          pl.BlockSpec((1, h), lambda i: (0, 0)),
          pl.BlockSpec((f, h), lambda i: (0, 0)),
          pl.BlockSpec((f, h), lambda i: (0, 0)),
          pl.BlockSpec((h, c), lambda i: (0, 0)),
          pl.BlockSpec((h, c), lambda i: (0, 0)),
      ],
      out_specs=[
          pl.BlockSpec((_RB, c), lambda i: (i, 0)),
          pl.BlockSpec((_RB, c), lambda i: (i, 0)),
      ],
      out_shape=[
          jax.ShapeDtypeStruct((n, c), jnp.float32),
          jax.ShapeDtypeStruct((n, c), jnp.float32),
      ],
  )


def _final_tc(n, c):
  def body(hs_ref, q_ref, degp_ref, b2_ref, out_ref):
    deg = degp_ref[0, :, :1] + degp_ref[1, :, :1]
    dinv = 1.0 / jnp.maximum(deg, 1.0)
    out_ref[...] = (hs_ref[...]
                    + (q_ref[0] + q_ref[1]) * dinv
                    + b2_ref[0])

  return pl.pallas_call(
      body,
      grid=(n // _RB,),
      in_specs=[
          pl.BlockSpec((_RB, c), lambda i: (i, 0)),
          pl.BlockSpec((_NC, _RB, c), lambda i: (0, i, 0)),
          pl.BlockSpec((_NC, _RB, _DW), lambda i: (0, i, 0)),
          pl.BlockSpec((1, c), lambda i: (0, 0)),
      ],
      out_specs=pl.BlockSpec((_RB, c), lambda i: (i, 0)),
      out_shape=jax.ShapeDtypeStruct((n, c), jnp.float32),
  )


@jax.jit
def kernel(in_feat, edge_index, W_self1, W_neigh1, b1, W_self2, W_neigh2, b2):
  n, f = in_feat.shape
  h = W_self1.shape[1]
  c = W_self2.shape[1]
  e = edge_index.shape[1]

  zr = n // _NS // 5
  zeros1 = jnp.zeros((zr, f), jnp.float32)
  zeros2 = jnp.zeros((zr, c), jnp.float32)
  aux = jnp.concatenate([jnp.zeros((128, _DW), jnp.float32),
                         jnp.ones((_CH, _DW), jnp.float32)])

  parts1, degp = _segsum_sc(n, e, f, with_deg=True)(
      in_feat, edge_index, zeros1, aux)
  p2, hs2 = _mid_tc(n, f, h, c)(
      in_feat, parts1, degp, b1.reshape(1, h), W_self1, W_neigh1,
      W_neigh2, W_self2)
  parts2, = _segsum_sc(n, e, c)(p2, edge_index, zeros2)
  return _final_tc(n, c)(hs2, parts2, degp, b2.reshape(1, c))
